# baseline (speedup 1.0000x reference)
import functools

import jax
import jax.numpy as jnp
from jax import lax
from jax.experimental import pallas as pl
from jax.experimental.pallas import tpu as pltpu
from jax.experimental.pallas import tpu_sc as plsc

H = 128
G3 = 3 * H
T = 2
L = 4
D = 128
N = 10000
NP = 10240
BN = 1000
GRID = N // BN
CH = 128
NL = 16
NCORES = 2
NSUB = 16
NTILES = NCORES * NSUB
RPT = NP // NSUB


def _gru_tail(gi, gh, h):
    r = jax.nn.sigmoid(gi[:, :H] + gh[:, :H])
    z = jax.nn.sigmoid(gi[:, H:2 * H] + gh[:, H:2 * H])
    n = jnp.tanh(gi[:, 2 * H:] + r * gh[:, 2 * H:])
    return (1.0 - z) * n + z * h


def _k1_body(c_ref, wihc_ref, whhc_ref, bihc_ref, bhhc_ref,
             wihu_ref, bihu_ref, bhhu_ref, hpre0_ref, hc1_ref):
    wihc = wihc_ref[...]
    whhc = whhc_ref[...]
    bihc = bihc_ref[...]
    bhhc = bhhc_ref[...]

    def encode(t):
        hc = jnp.zeros((BN, H), jnp.float32)
        for l in range(L):
            x = c_ref[t, :, l, :]
            gi = jnp.dot(x, wihc, preferred_element_type=jnp.float32) + bihc
            gh = jnp.dot(hc, whhc, preferred_element_type=jnp.float32) + bhhc
            hc = _gru_tail(gi, gh, hc)
        return hc

    hc0 = encode(0)
    hc1 = encode(1)
    gi = jnp.dot(hc0, wihu_ref[...], preferred_element_type=jnp.float32) + bihu_ref[...]
    gh = jnp.broadcast_to(bhhu_ref[...], (BN, G3))
    hpre0_ref[...] = _gru_tail(gi, gh, jnp.zeros((BN, H), jnp.float32))
    hc1_ref[...] = hc1


def _sage(hpre, s, cnt, ws, bsb, wn, bnb):
    mean = s / jnp.maximum(cnt, 1.0)
    self_msg = jnp.dot(hpre, ws, preferred_element_type=jnp.float32) + bsb
    neigh = jnp.dot(mean, wn, preferred_element_type=jnp.float32) + bnb
    neigh = jnp.where(cnt > 0.0, neigh, 0.0)
    return jax.nn.relu(self_msg + neigh)


def _k4_body(hpre0_ref, part_ref, cntp_ref, hc1_ref, tf_ref,
             ws_ref, bs_ref, wn_ref, bn_ref, wc_ref, bc_ref,
             wihu_ref, wulast_ref, whhu_ref, bihu_ref, bhhu_ref,
             l0_ref, hpre1_ref):
    hpre0 = hpre0_ref[...]
    s = part_ref[0] + part_ref[1]
    cnt = jnp.sum(cntp_ref[...], axis=0)
    h0 = _sage(hpre0, s, cnt, ws_ref[...], bs_ref[...], wn_ref[...], bn_ref[...])
    l0_ref[...] = jnp.dot(h0, wc_ref[...], preferred_element_type=jnp.float32) + bc_ref[...]
    prev_y = tf_ref[:, 0:1]
    gi = (jnp.dot(hc1_ref[...], wihu_ref[...], preferred_element_type=jnp.float32)
          + prev_y * wulast_ref[...] + bihu_ref[...])
    gh = jnp.dot(h0, whhu_ref[...], preferred_element_type=jnp.float32) + bhhu_ref[...]
    hpre1_ref[...] = _gru_tail(gi, gh, h0)


def _k6_body(hpre1_ref, part_ref, cntp_ref,
             ws_ref, bs_ref, wn_ref, bn_ref, wc_ref, bc_ref, l1_ref):
    s = part_ref[0] + part_ref[1]
    cnt = jnp.sum(cntp_ref[...], axis=0)
    h1 = _sage(hpre1_ref[...], s, cnt, ws_ref[...], bs_ref[...],
               wn_ref[...], bn_ref[...])
    l1_ref[...] = jnp.dot(h1, wc_ref[...], preferred_element_type=jnp.float32) + bc_ref[...]


def _rep(shape):
    nd = len(shape)
    return pl.BlockSpec(shape, lambda i, _n=nd: (0,) * _n)


def _make_sc_agg(cpt, with_cnt, ch=CH):
    mesh = plsc.VectorSubcoreMesh(core_axis_name="c", subcore_axis_name="s")
    outs = [jax.ShapeDtypeStruct((NCORES * NP, H), jnp.float32)]
    scratch = [
        pltpu.VMEM((ch,), jnp.int32),
        pltpu.VMEM((ch,), jnp.int32),
        pltpu.VMEM((ch, H), jnp.float32),
        pltpu.VMEM_SHARED((NP, H), jnp.float32),
        pltpu.SemaphoreType.DMA,
    ]
    if with_cnt:
        outs.append(jax.ShapeDtypeStruct((NTILES * NP,), jnp.float32))
        scratch.append(pltpu.VMEM((NP,), jnp.float32))

    def body(h_hbm, src_hbm, dst_hbm, zrow_hbm, part_out,
             dst_v, src_v, rows_v, accum, sem,
             zcnt_hbm=None, cnt_out=None, cnt_v=None):
        cid = lax.axis_index("c")
        sid = lax.axis_index("s")
        wid = cid * NSUB + sid
        r0 = sid * RPT
        pltpu.sync_copy(zrow_hbm, accum.at[pl.ds(r0, RPT)])
        if with_cnt:
            pltpu.sync_copy(zcnt_hbm, cnt_v)
        plsc.subcore_barrier()
        ones16 = jnp.ones((NL,), jnp.float32)

        def chunk(j, carry):
            base = (wid * cpt + j) * ch
            pltpu.sync_copy(src_hbm.at[pl.ds(base, ch)], src_v)
            pltpu.sync_copy(dst_hbm.at[pl.ds(base, ch)], dst_v)
            pltpu.async_copy(h_hbm.at[dst_v], rows_v, sem).wait()
            pltpu.sync_copy(rows_v, accum.at[src_v], add=True)
            if with_cnt:
                for i in range(ch // NL):
                    idx16 = src_v[pl.ds(i * NL, NL)]
                    plsc.addupdate_scatter(cnt_v, [idx16], ones16)
            return carry

        lax.fori_loop(0, cpt, chunk, 0)
        plsc.subcore_barrier()
        pltpu.sync_copy(accum.at[pl.ds(r0, RPT)],
                        part_out.at[pl.ds(cid * NP + r0, RPT)])
        if with_cnt:
            pltpu.sync_copy(cnt_v, cnt_out.at[pl.ds(wid * NP, NP)])

    cp = pltpu.CompilerParams(needs_layout_passes=False) if with_cnt else None
    if with_cnt:
        @functools.partial(pl.kernel, mesh=mesh, out_type=outs,
                           scratch_types=scratch, compiler_params=cp)
        def k(h_hbm, src_hbm, dst_hbm, zrow_hbm, zcnt_hbm, part_out, cnt_out,
              dst_v, src_v, rows_v, accum, sem, cnt_v):
            body(h_hbm, src_hbm, dst_hbm, zrow_hbm, part_out,
                 dst_v, src_v, rows_v, accum, sem,
                 zcnt_hbm=zcnt_hbm, cnt_out=cnt_out, cnt_v=cnt_v)
    else:
        @functools.partial(pl.kernel, mesh=mesh, out_type=outs,
                           scratch_types=scratch)
        def k(h_hbm, src_hbm, dst_hbm, zrow_hbm, part_out,
              dst_v, src_v, rows_v, accum, sem):
            body(h_hbm, src_hbm, dst_hbm, zrow_hbm, part_out,
                 dst_v, src_v, rows_v, accum, sem)
    return k


def _build_k1():
    return pl.pallas_call(
        _k1_body,
        grid=(GRID,),
        in_specs=[
            pl.BlockSpec((T, BN, L, D), lambda i: (0, i, 0, 0)),
            _rep((D, G3)), _rep((H, G3)), _rep((1, G3)), _rep((1, G3)),
            _rep((H, G3)), _rep((1, G3)), _rep((1, G3)),
        ],
        out_specs=[pl.BlockSpec((BN, H), lambda i: (i, 0)),
                   pl.BlockSpec((BN, H), lambda i: (i, 0))],
        out_shape=[jax.ShapeDtypeStruct((N, H), jnp.float32),
                   jax.ShapeDtypeStruct((N, H), jnp.float32)],
    )


def _build_k4():
    return pl.pallas_call(
        _k4_body,
        grid=(GRID,),
        in_specs=[
            pl.BlockSpec((BN, H), lambda i: (i, 0)),
            pl.BlockSpec((NCORES, BN, H), lambda i: (0, i, 0)),
            pl.BlockSpec((NTILES, BN, 1), lambda i: (0, i, 0)),
            pl.BlockSpec((BN, H), lambda i: (i, 0)),
            pl.BlockSpec((BN, T), lambda i: (i, 0)),
            _rep((H, H)), _rep((1, H)), _rep((H, H)), _rep((1, H)),
            _rep((H, 1)), _rep((1, 1)),
            _rep((H, G3)), _rep((1, G3)), _rep((H, G3)),
            _rep((1, G3)), _rep((1, G3)),
        ],
        out_specs=[pl.BlockSpec((BN, 1), lambda i: (i, 0)),
                   pl.BlockSpec((BN, H), lambda i: (i, 0))],
        out_shape=[jax.ShapeDtypeStruct((N, 1), jnp.float32),
                   jax.ShapeDtypeStruct((N, H), jnp.float32)],
    )


def _build_k6():
    return pl.pallas_call(
        _k6_body,
        grid=(GRID,),
        in_specs=[
            pl.BlockSpec((BN, H), lambda i: (i, 0)),
            pl.BlockSpec((NCORES, BN, H), lambda i: (0, i, 0)),
            pl.BlockSpec((NTILES, BN, 1), lambda i: (0, i, 0)),
            _rep((H, H)), _rep((1, H)), _rep((H, H)), _rep((1, H)),
            _rep((H, 1)), _rep((1, 1)),
        ],
        out_specs=pl.BlockSpec((BN, 1), lambda i: (i, 0)),
        out_shape=jax.ShapeDtypeStruct((N, 1), jnp.float32),
    )


def kernel(comments, truth_flags, edge_index, W_ih_c, W_hh_c, b_ih_c, b_hh_c,
           W_ih_u, W_hh_u, b_ih_u, b_hh_u, Ws, bs, Wn, bn, Wc, bc):
    f32 = jnp.float32
    wihc = W_ih_c.T
    whhc = W_hh_c.T
    bihc = b_ih_c.reshape(1, G3)
    bhhc = b_hh_c.reshape(1, G3)
    wihu = W_ih_u[:, :H].T
    wulast = W_ih_u[:, H].reshape(1, G3)
    whhu = W_hh_u.T
    bihu = b_ih_u.reshape(1, G3)
    bhhu = b_hh_u.reshape(1, G3)
    ws_t = Ws.T
    wn_t = Wn.T
    bs2 = bs.reshape(1, H)
    bn2 = bn.reshape(1, H)
    wc_t = Wc.T
    bc2 = bc.reshape(1, 1)
    tf_t = truth_flags.T.astype(f32)

    src = edge_index[0].astype(jnp.int32)
    dst = edge_index[1].astype(jnp.int32)
    e = src.shape[0]
    span = NTILES * CH * 2
    ep = ((e + span - 1) // span) * span
    cpt = ep // (NTILES * CH)
    pad = ep - e
    if pad:
        src = jnp.concatenate([src, jnp.full((pad,), NP - 1, jnp.int32)])
        dst = jnp.concatenate([dst, jnp.zeros((pad,), jnp.int32)])

    zrow = jnp.zeros((RPT, H), f32)
    zcnt = jnp.zeros((NP,), f32)

    hpre0, hc1 = _build_k1()(comments, wihc, whhc, bihc, bhhc, wihu, bihu, bhhu)

    part0f, cntf = _make_sc_agg(cpt, with_cnt=True)(hpre0, src, dst, zrow, zcnt)
    part0 = part0f.reshape(NCORES, NP, H)
    cntp = cntf.reshape(NTILES, NP, 1)

    l0, hpre1 = _build_k4()(hpre0, part0, cntp, hc1, tf_t,
                            ws_t, bs2, wn_t, bn2, wc_t, bc2,
                            wihu, wulast, whhu, bihu, bhhu)

    (part1f,) = _make_sc_agg(cpt, with_cnt=False)(hpre1, src, dst, zrow)
    part1 = part1f.reshape(NCORES, NP, H)

    l1 = _build_k6()(hpre1, part1, cntp, ws_t, bs2, wn_t, bn2, wc_t, bc2)

    logits = jnp.concatenate([l0.T, l1.T], axis=0)
    return (logits, truth_flags)


# speedup vs baseline: 3.6300x; 3.6300x over previous
import functools

import jax
import jax.numpy as jnp
from jax import lax
from jax.experimental import pallas as pl
from jax.experimental.pallas import tpu as pltpu
from jax.experimental.pallas import tpu_sc as plsc

H = 128
G3 = 3 * H
T = 2
L = 4
D = 128
N = 10000
NP = 10240
BN = 1000
GRID = N // BN
CH = 128
NL = 16
NCORES = 2
NSUB = 16
NTILES = NCORES * NSUB
RPT = NP // NSUB


def _gru_tail(gi, gh, h):
    r = jax.nn.sigmoid(gi[:, :H] + gh[:, :H])
    z = jax.nn.sigmoid(gi[:, H:2 * H] + gh[:, H:2 * H])
    n = jnp.tanh(gi[:, 2 * H:] + r * gh[:, 2 * H:])
    return (1.0 - z) * n + z * h


def _k1_body(c_ref, wihc_ref, whhc_ref, bihc_ref, bhhc_ref,
             wihu_ref, bihu_ref, bhhu_ref, hpre0_ref, hc1_ref):
    wihc = wihc_ref[...]
    whhc = whhc_ref[...]
    bihc = bihc_ref[...]
    bhhc = bhhc_ref[...]

    def encode(t):
        hc = jnp.zeros((BN, H), jnp.float32)
        for l in range(L):
            x = c_ref[t, :, l, :]
            gi = jnp.dot(x, wihc, preferred_element_type=jnp.float32) + bihc
            gh = jnp.dot(hc, whhc, preferred_element_type=jnp.float32) + bhhc
            hc = _gru_tail(gi, gh, hc)
        return hc

    hc0 = encode(0)
    hc1 = encode(1)
    gi = jnp.dot(hc0, wihu_ref[...], preferred_element_type=jnp.float32) + bihu_ref[...]
    gh = jnp.broadcast_to(bhhu_ref[...], (BN, G3))
    hpre0_ref[...] = _gru_tail(gi, gh, jnp.zeros((BN, H), jnp.float32))
    hc1_ref[...] = hc1


def _sage(hpre, s, cnt, ws, bsb, wn, bnb):
    mean = s / jnp.maximum(cnt, 1.0)
    self_msg = jnp.dot(hpre, ws, preferred_element_type=jnp.float32) + bsb
    neigh = jnp.dot(mean, wn, preferred_element_type=jnp.float32) + bnb
    neigh = jnp.where(cnt > 0.0, neigh, 0.0)
    return jax.nn.relu(self_msg + neigh)


def _k4_body(hpre0_ref, part_ref, cntp_ref, hc1_ref, tf_ref,
             ws_ref, bs_ref, wn_ref, bn_ref, wc_ref, bc_ref,
             wihu_ref, wulast_ref, whhu_ref, bihu_ref, bhhu_ref,
             l0_ref, hpre1_ref):
    hpre0 = hpre0_ref[...]
    s = part_ref[0] + part_ref[1]
    cnt = jnp.sum(cntp_ref[...], axis=0)
    h0 = _sage(hpre0, s, cnt, ws_ref[...], bs_ref[...], wn_ref[...], bn_ref[...])
    l0_ref[...] = jnp.dot(h0, wc_ref[...], preferred_element_type=jnp.float32) + bc_ref[...]
    prev_y = tf_ref[:, 0:1]
    gi = (jnp.dot(hc1_ref[...], wihu_ref[...], preferred_element_type=jnp.float32)
          + prev_y * wulast_ref[...] + bihu_ref[...])
    gh = jnp.dot(h0, whhu_ref[...], preferred_element_type=jnp.float32) + bhhu_ref[...]
    hpre1_ref[...] = _gru_tail(gi, gh, h0)


def _k6_body(hpre1_ref, part_ref, cntp_ref,
             ws_ref, bs_ref, wn_ref, bn_ref, wc_ref, bc_ref, l1_ref):
    s = part_ref[0] + part_ref[1]
    cnt = jnp.sum(cntp_ref[...], axis=0)
    h1 = _sage(hpre1_ref[...], s, cnt, ws_ref[...], bs_ref[...],
               wn_ref[...], bn_ref[...])
    l1_ref[...] = jnp.dot(h1, wc_ref[...], preferred_element_type=jnp.float32) + bc_ref[...]


def _rep(shape):
    nd = len(shape)
    return pl.BlockSpec(shape, lambda i, _n=nd: (0,) * _n)


def _make_sc_agg(cpt, with_cnt, ch=CH):
    mesh = plsc.VectorSubcoreMesh(core_axis_name="c", subcore_axis_name="s")
    outs = [jax.ShapeDtypeStruct((NCORES * NP, H), jnp.float32)]
    scratch = [
        pltpu.VMEM((ch,), jnp.int32),
        pltpu.VMEM((ch,), jnp.int32),
        pltpu.VMEM((ch, H), jnp.float32),
        pltpu.VMEM_SHARED((NP, H), jnp.float32),
        pltpu.SemaphoreType.DMA,
    ]
    if with_cnt:
        outs.append(jax.ShapeDtypeStruct((NTILES * NP,), jnp.float32))
        scratch.append(pltpu.VMEM((NP,), jnp.float32))

    def body(h_hbm, src_hbm, dst_hbm, zrow_hbm, part_out,
             dst_v, src_v, rows_v, accum, sem,
             zcnt_hbm=None, cnt_out=None, cnt_v=None):
        cid = lax.axis_index("c")
        sid = lax.axis_index("s")
        wid = cid * NSUB + sid
        r0 = sid * RPT
        pltpu.sync_copy(zrow_hbm, accum.at[pl.ds(r0, RPT)])
        if with_cnt:
            pltpu.sync_copy(zcnt_hbm, cnt_v)
        plsc.subcore_barrier()
        ones16 = jnp.ones((NL,), jnp.float32)

        @functools.partial(plsc.parallel_loop, 0, cpt)
        def _(j):
            base = (wid * cpt + j) * ch
            pltpu.sync_copy(src_hbm.at[pl.ds(base, ch)], src_v)
            pltpu.sync_copy(dst_hbm.at[pl.ds(base, ch)], dst_v)
            pltpu.async_copy(h_hbm.at[dst_v], rows_v, sem).wait()
            pltpu.sync_copy(rows_v, accum.at[src_v], add=True)
            if with_cnt:
                for i in range(ch // NL):
                    idx16 = src_v[pl.ds(i * NL, NL)]
                    plsc.addupdate_scatter(cnt_v, [idx16], ones16)
        plsc.subcore_barrier()
        pltpu.sync_copy(accum.at[pl.ds(r0, RPT)],
                        part_out.at[pl.ds(cid * NP + r0, RPT)])
        if with_cnt:
            pltpu.sync_copy(cnt_v, cnt_out.at[pl.ds(wid * NP, NP)])

    cp = pltpu.CompilerParams(needs_layout_passes=False) if with_cnt else None
    if with_cnt:
        @functools.partial(pl.kernel, mesh=mesh, out_type=outs,
                           scratch_types=scratch, compiler_params=cp)
        def k(h_hbm, src_hbm, dst_hbm, zrow_hbm, zcnt_hbm, part_out, cnt_out,
              dst_v, src_v, rows_v, accum, sem, cnt_v):
            body(h_hbm, src_hbm, dst_hbm, zrow_hbm, part_out,
                 dst_v, src_v, rows_v, accum, sem,
                 zcnt_hbm=zcnt_hbm, cnt_out=cnt_out, cnt_v=cnt_v)
    else:
        @functools.partial(pl.kernel, mesh=mesh, out_type=outs,
                           scratch_types=scratch)
        def k(h_hbm, src_hbm, dst_hbm, zrow_hbm, part_out,
              dst_v, src_v, rows_v, accum, sem):
            body(h_hbm, src_hbm, dst_hbm, zrow_hbm, part_out,
                 dst_v, src_v, rows_v, accum, sem)
    return k


def _build_k1():
    return pl.pallas_call(
        _k1_body,
        grid=(GRID,),
        in_specs=[
            pl.BlockSpec((T, BN, L, D), lambda i: (0, i, 0, 0)),
            _rep((D, G3)), _rep((H, G3)), _rep((1, G3)), _rep((1, G3)),
            _rep((H, G3)), _rep((1, G3)), _rep((1, G3)),
        ],
        out_specs=[pl.BlockSpec((BN, H), lambda i: (i, 0)),
                   pl.BlockSpec((BN, H), lambda i: (i, 0))],
        out_shape=[jax.ShapeDtypeStruct((N, H), jnp.float32),
                   jax.ShapeDtypeStruct((N, H), jnp.float32)],
    )


def _build_k4():
    return pl.pallas_call(
        _k4_body,
        grid=(GRID,),
        in_specs=[
            pl.BlockSpec((BN, H), lambda i: (i, 0)),
            pl.BlockSpec((NCORES, BN, H), lambda i: (0, i, 0)),
            pl.BlockSpec((NTILES, BN, 1), lambda i: (0, i, 0)),
            pl.BlockSpec((BN, H), lambda i: (i, 0)),
            pl.BlockSpec((BN, T), lambda i: (i, 0)),
            _rep((H, H)), _rep((1, H)), _rep((H, H)), _rep((1, H)),
            _rep((H, 1)), _rep((1, 1)),
            _rep((H, G3)), _rep((1, G3)), _rep((H, G3)),
            _rep((1, G3)), _rep((1, G3)),
        ],
        out_specs=[pl.BlockSpec((BN, 1), lambda i: (i, 0)),
                   pl.BlockSpec((BN, H), lambda i: (i, 0))],
        out_shape=[jax.ShapeDtypeStruct((N, 1), jnp.float32),
                   jax.ShapeDtypeStruct((N, H), jnp.float32)],
    )


def _build_k6():
    return pl.pallas_call(
        _k6_body,
        grid=(GRID,),
        in_specs=[
            pl.BlockSpec((BN, H), lambda i: (i, 0)),
            pl.BlockSpec((NCORES, BN, H), lambda i: (0, i, 0)),
            pl.BlockSpec((NTILES, BN, 1), lambda i: (0, i, 0)),
            _rep((H, H)), _rep((1, H)), _rep((H, H)), _rep((1, H)),
            _rep((H, 1)), _rep((1, 1)),
        ],
        out_specs=pl.BlockSpec((BN, 1), lambda i: (i, 0)),
        out_shape=jax.ShapeDtypeStruct((N, 1), jnp.float32),
    )


def kernel(comments, truth_flags, edge_index, W_ih_c, W_hh_c, b_ih_c, b_hh_c,
           W_ih_u, W_hh_u, b_ih_u, b_hh_u, Ws, bs, Wn, bn, Wc, bc):
    f32 = jnp.float32
    wihc = W_ih_c.T
    whhc = W_hh_c.T
    bihc = b_ih_c.reshape(1, G3)
    bhhc = b_hh_c.reshape(1, G3)
    wihu = W_ih_u[:, :H].T
    wulast = W_ih_u[:, H].reshape(1, G3)
    whhu = W_hh_u.T
    bihu = b_ih_u.reshape(1, G3)
    bhhu = b_hh_u.reshape(1, G3)
    ws_t = Ws.T
    wn_t = Wn.T
    bs2 = bs.reshape(1, H)
    bn2 = bn.reshape(1, H)
    wc_t = Wc.T
    bc2 = bc.reshape(1, 1)
    tf_t = truth_flags.T.astype(f32)

    src = edge_index[0].astype(jnp.int32)
    dst = edge_index[1].astype(jnp.int32)
    e = src.shape[0]
    span = NTILES * CH * 2
    ep = ((e + span - 1) // span) * span
    cpt = ep // (NTILES * CH)
    pad = ep - e
    if pad:
        src = jnp.concatenate([src, jnp.full((pad,), NP - 1, jnp.int32)])
        dst = jnp.concatenate([dst, jnp.zeros((pad,), jnp.int32)])

    zrow = jnp.zeros((RPT, H), f32)
    zcnt = jnp.zeros((NP,), f32)

    hpre0, hc1 = _build_k1()(comments, wihc, whhc, bihc, bhhc, wihu, bihu, bhhu)

    part0f, cntf = _make_sc_agg(cpt, with_cnt=True)(hpre0, src, dst, zrow, zcnt)
    part0 = part0f.reshape(NCORES, NP, H)
    cntp = cntf.reshape(NTILES, NP, 1)

    l0, hpre1 = _build_k4()(hpre0, part0, cntp, hc1, tf_t,
                            ws_t, bs2, wn_t, bn2, wc_t, bc2,
                            wihu, wulast, whhu, bihu, bhhu)

    (part1f,) = _make_sc_agg(cpt, with_cnt=False)(hpre1, src, dst, zrow)
    part1 = part1f.reshape(NCORES, NP, H)

    l1 = _build_k6()(hpre1, part1, cntp, ws_t, bs2, wn_t, bn2, wc_t, bc2)

    logits = jnp.concatenate([l0.T, l1.T], axis=0)
    return (logits, truth_flags)


# speedup vs baseline: 3.6355x; 1.0015x over previous
import functools

import jax
import jax.numpy as jnp
from jax import lax
from jax.experimental import pallas as pl
from jax.experimental.pallas import tpu as pltpu
from jax.experimental.pallas import tpu_sc as plsc

H = 128
G3 = 3 * H
T = 2
L = 4
D = 128
N = 10000
NP = 10240
BN = 1000
GRID = N // BN
CH = 128
NL = 16
NCORES = 2
NSUB = 16
NTILES = NCORES * NSUB
RPT = NP // NSUB


def _gru_tail(gi, gh, h):
    r = jax.nn.sigmoid(gi[:, :H] + gh[:, :H])
    z = jax.nn.sigmoid(gi[:, H:2 * H] + gh[:, H:2 * H])
    n = jnp.tanh(gi[:, 2 * H:] + r * gh[:, 2 * H:])
    return (1.0 - z) * n + z * h


def _k1_body(c_ref, wihc_ref, whhc_ref, bihc_ref, bhhc_ref,
             wihu_ref, bihu_ref, bhhu_ref, hpre0_ref, hc1_ref):
    wihc = wihc_ref[...]
    whhc = whhc_ref[...]
    bihc = bihc_ref[...]
    bhhc = bhhc_ref[...]

    def encode(t):
        hc = jnp.zeros((BN, H), jnp.float32)
        for l in range(L):
            x = c_ref[t, :, l, :]
            gi = jnp.dot(x, wihc, preferred_element_type=jnp.float32) + bihc
            gh = jnp.dot(hc, whhc, preferred_element_type=jnp.float32) + bhhc
            hc = _gru_tail(gi, gh, hc)
        return hc

    hc0 = encode(0)
    hc1 = encode(1)
    gi = jnp.dot(hc0, wihu_ref[...], preferred_element_type=jnp.float32) + bihu_ref[...]
    gh = jnp.broadcast_to(bhhu_ref[...], (BN, G3))
    hpre0_ref[...] = _gru_tail(gi, gh, jnp.zeros((BN, H), jnp.float32))
    hc1_ref[...] = hc1


def _sage(hpre, s, cnt, ws, bsb, wn, bnb):
    mean = s / jnp.maximum(cnt, 1.0)
    self_msg = jnp.dot(hpre, ws, preferred_element_type=jnp.float32) + bsb
    neigh = jnp.dot(mean, wn, preferred_element_type=jnp.float32) + bnb
    neigh = jnp.where(cnt > 0.0, neigh, 0.0)
    return jax.nn.relu(self_msg + neigh)


def _k4_body(hpre0_ref, part_ref, cntp_ref, hc1_ref, tf_ref,
             ws_ref, bs_ref, wn_ref, bn_ref, wc_ref, bc_ref,
             wihu_ref, wulast_ref, whhu_ref, bihu_ref, bhhu_ref,
             l0_ref, hpre1_ref):
    hpre0 = hpre0_ref[...]
    s = part_ref[0] + part_ref[1]
    cnt = jnp.sum(cntp_ref[...], axis=0)
    h0 = _sage(hpre0, s, cnt, ws_ref[...], bs_ref[...], wn_ref[...], bn_ref[...])
    l0_ref[...] = jnp.dot(h0, wc_ref[...], preferred_element_type=jnp.float32) + bc_ref[...]
    prev_y = tf_ref[:, 0:1]
    gi = (jnp.dot(hc1_ref[...], wihu_ref[...], preferred_element_type=jnp.float32)
          + prev_y * wulast_ref[...] + bihu_ref[...])
    gh = jnp.dot(h0, whhu_ref[...], preferred_element_type=jnp.float32) + bhhu_ref[...]
    hpre1_ref[...] = _gru_tail(gi, gh, h0)


def _k6_body(hpre1_ref, part_ref, cntp_ref,
             ws_ref, bs_ref, wn_ref, bn_ref, wc_ref, bc_ref, l1_ref):
    s = part_ref[0] + part_ref[1]
    cnt = jnp.sum(cntp_ref[...], axis=0)
    h1 = _sage(hpre1_ref[...], s, cnt, ws_ref[...], bs_ref[...],
               wn_ref[...], bn_ref[...])
    l1_ref[...] = jnp.dot(h1, wc_ref[...], preferred_element_type=jnp.float32) + bc_ref[...]


def _rep(shape):
    nd = len(shape)
    return pl.BlockSpec(shape, lambda i, _n=nd: (0,) * _n)


def _make_sc_agg(cpt, with_cnt, ch=CH):
    mesh = plsc.VectorSubcoreMesh(core_axis_name="c", subcore_axis_name="s")
    outs = [jax.ShapeDtypeStruct((NCORES * NP, H), jnp.float32)]
    scratch = [
        pltpu.VMEM((2, ch), jnp.int32),
        pltpu.VMEM((2, ch), jnp.int32),
        pltpu.VMEM((2, ch, H), jnp.float32),
        pltpu.VMEM_SHARED((NP, H), jnp.float32),
        pltpu.SemaphoreType.DMA,
    ]
    if with_cnt:
        outs.append(jax.ShapeDtypeStruct((NTILES * NP,), jnp.float32))
        scratch.append(pltpu.VMEM((NP,), jnp.float32))

    def body(h_hbm, src_hbm, dst_hbm, zrow_hbm, part_out,
             dst_v, src_v, rows_v, accum, sem,
             zcnt_hbm=None, cnt_out=None, cnt_v=None):
        cid = lax.axis_index("c")
        sid = lax.axis_index("s")
        wid = cid * NSUB + sid
        r0 = sid * RPT
        pltpu.sync_copy(zrow_hbm, accum.at[pl.ds(r0, RPT)])
        if with_cnt:
            pltpu.sync_copy(zcnt_hbm, cnt_v)
        plsc.subcore_barrier()
        ones16 = jnp.ones((NL,), jnp.float32)

        @functools.partial(plsc.parallel_loop, 0, cpt, unroll=2)
        def _(j):
            b = j % 2
            sv = src_v.at[b]
            dv = dst_v.at[b]
            rv = rows_v.at[b]
            base = (wid * cpt + j) * ch
            pltpu.sync_copy(src_hbm.at[pl.ds(base, ch)], sv)
            pltpu.sync_copy(dst_hbm.at[pl.ds(base, ch)], dv)
            pltpu.async_copy(h_hbm.at[dv], rv, sem).wait()
            pltpu.sync_copy(rv, accum.at[sv], add=True)
            if with_cnt:
                for i in range(ch // NL):
                    idx16 = src_v[b, pl.ds(i * NL, NL)]
                    plsc.addupdate_scatter(cnt_v, [idx16], ones16)
        plsc.subcore_barrier()
        pltpu.sync_copy(accum.at[pl.ds(r0, RPT)],
                        part_out.at[pl.ds(cid * NP + r0, RPT)])
        if with_cnt:
            pltpu.sync_copy(cnt_v, cnt_out.at[pl.ds(wid * NP, NP)])

    cp = pltpu.CompilerParams(needs_layout_passes=False) if with_cnt else None
    if with_cnt:
        @functools.partial(pl.kernel, mesh=mesh, out_type=outs,
                           scratch_types=scratch, compiler_params=cp)
        def k(h_hbm, src_hbm, dst_hbm, zrow_hbm, zcnt_hbm, part_out, cnt_out,
              dst_v, src_v, rows_v, accum, sem, cnt_v):
            body(h_hbm, src_hbm, dst_hbm, zrow_hbm, part_out,
                 dst_v, src_v, rows_v, accum, sem,
                 zcnt_hbm=zcnt_hbm, cnt_out=cnt_out, cnt_v=cnt_v)
    else:
        @functools.partial(pl.kernel, mesh=mesh, out_type=outs,
                           scratch_types=scratch)
        def k(h_hbm, src_hbm, dst_hbm, zrow_hbm, part_out,
              dst_v, src_v, rows_v, accum, sem):
            body(h_hbm, src_hbm, dst_hbm, zrow_hbm, part_out,
                 dst_v, src_v, rows_v, accum, sem)
    return k


def _build_k1():
    return pl.pallas_call(
        _k1_body,
        grid=(GRID,),
        in_specs=[
            pl.BlockSpec((T, BN, L, D), lambda i: (0, i, 0, 0)),
            _rep((D, G3)), _rep((H, G3)), _rep((1, G3)), _rep((1, G3)),
            _rep((H, G3)), _rep((1, G3)), _rep((1, G3)),
        ],
        out_specs=[pl.BlockSpec((BN, H), lambda i: (i, 0)),
                   pl.BlockSpec((BN, H), lambda i: (i, 0))],
        out_shape=[jax.ShapeDtypeStruct((N, H), jnp.float32),
                   jax.ShapeDtypeStruct((N, H), jnp.float32)],
    )


def _build_k4():
    return pl.pallas_call(
        _k4_body,
        grid=(GRID,),
        in_specs=[
            pl.BlockSpec((BN, H), lambda i: (i, 0)),
            pl.BlockSpec((NCORES, BN, H), lambda i: (0, i, 0)),
            pl.BlockSpec((NTILES, BN, 1), lambda i: (0, i, 0)),
            pl.BlockSpec((BN, H), lambda i: (i, 0)),
            pl.BlockSpec((BN, T), lambda i: (i, 0)),
            _rep((H, H)), _rep((1, H)), _rep((H, H)), _rep((1, H)),
            _rep((H, 1)), _rep((1, 1)),
            _rep((H, G3)), _rep((1, G3)), _rep((H, G3)),
            _rep((1, G3)), _rep((1, G3)),
        ],
        out_specs=[pl.BlockSpec((BN, 1), lambda i: (i, 0)),
                   pl.BlockSpec((BN, H), lambda i: (i, 0))],
        out_shape=[jax.ShapeDtypeStruct((N, 1), jnp.float32),
                   jax.ShapeDtypeStruct((N, H), jnp.float32)],
    )


def _build_k6():
    return pl.pallas_call(
        _k6_body,
        grid=(GRID,),
        in_specs=[
            pl.BlockSpec((BN, H), lambda i: (i, 0)),
            pl.BlockSpec((NCORES, BN, H), lambda i: (0, i, 0)),
            pl.BlockSpec((NTILES, BN, 1), lambda i: (0, i, 0)),
            _rep((H, H)), _rep((1, H)), _rep((H, H)), _rep((1, H)),
            _rep((H, 1)), _rep((1, 1)),
        ],
        out_specs=pl.BlockSpec((BN, 1), lambda i: (i, 0)),
        out_shape=jax.ShapeDtypeStruct((N, 1), jnp.float32),
    )


def kernel(comments, truth_flags, edge_index, W_ih_c, W_hh_c, b_ih_c, b_hh_c,
           W_ih_u, W_hh_u, b_ih_u, b_hh_u, Ws, bs, Wn, bn, Wc, bc):
    f32 = jnp.float32
    wihc = W_ih_c.T
    whhc = W_hh_c.T
    bihc = b_ih_c.reshape(1, G3)
    bhhc = b_hh_c.reshape(1, G3)
    wihu = W_ih_u[:, :H].T
    wulast = W_ih_u[:, H].reshape(1, G3)
    whhu = W_hh_u.T
    bihu = b_ih_u.reshape(1, G3)
    bhhu = b_hh_u.reshape(1, G3)
    ws_t = Ws.T
    wn_t = Wn.T
    bs2 = bs.reshape(1, H)
    bn2 = bn.reshape(1, H)
    wc_t = Wc.T
    bc2 = bc.reshape(1, 1)
    tf_t = truth_flags.T.astype(f32)

    src = edge_index[0].astype(jnp.int32)
    dst = edge_index[1].astype(jnp.int32)
    e = src.shape[0]
    span = NTILES * CH * 2
    ep = ((e + span - 1) // span) * span
    cpt = ep // (NTILES * CH)
    pad = ep - e
    if pad:
        src = jnp.concatenate([src, jnp.full((pad,), NP - 1, jnp.int32)])
        dst = jnp.concatenate([dst, jnp.zeros((pad,), jnp.int32)])

    zrow = jnp.zeros((RPT, H), f32)
    zcnt = jnp.zeros((NP,), f32)

    hpre0, hc1 = _build_k1()(comments, wihc, whhc, bihc, bhhc, wihu, bihu, bhhu)

    part0f, cntf = _make_sc_agg(cpt, with_cnt=True)(hpre0, src, dst, zrow, zcnt)
    part0 = part0f.reshape(NCORES, NP, H)
    cntp = cntf.reshape(NTILES, NP, 1)

    l0, hpre1 = _build_k4()(hpre0, part0, cntp, hc1, tf_t,
                            ws_t, bs2, wn_t, bn2, wc_t, bc2,
                            wihu, wulast, whhu, bihu, bhhu)

    (part1f,) = _make_sc_agg(cpt, with_cnt=False)(hpre1, src, dst, zrow)
    part1 = part1f.reshape(NCORES, NP, H)

    l1 = _build_k6()(hpre1, part1, cntp, ws_t, bs2, wn_t, bn2, wc_t, bc2)

    logits = jnp.concatenate([l0.T, l1.T], axis=0)
    return (logits, truth_flags)
